# baseline (speedup 1.0000x reference)
import functools

import jax
import jax.numpy as jnp
from jax import lax
from jax.experimental import pallas as pl
from jax.experimental.pallas import tpu as pltpu
from jax.experimental.pallas import tpu_sc as plsc


@functools.cache
def _make_sc_embed(B: int, S: int, V: int, D: int):
    info = plsc.get_sparse_core_info()
    NC, NS, L = info.num_cores, info.num_subcores, info.num_lanes
    NW = NC * NS
    assert S % NW == 0
    s_per_w = S // NW
    CHUNK = 8
    assert s_per_w % CHUNK == 0
    n_groups = s_per_w // CHUNK
    NSLOT = 2
    mesh = plsc.VectorSubcoreMesh(core_axis_name="c", subcore_axis_name="s")

    @functools.partial(
        pl.kernel,
        mesh=mesh,
        out_type=jax.ShapeDtypeStruct((B * S, D), jnp.float32),
        scratch_types=[
            pltpu.VMEM((B * s_per_w,), jnp.int32),
            [[pltpu.VMEM((CHUNK, D), jnp.float32) for _ in range(B)]
             for _ in range(NSLOT)],
            [pltpu.VMEM((CHUNK, D), jnp.float32) for _ in range(NSLOT)],
            [pltpu.SemaphoreType.DMA for _ in range(NSLOT)],
            [pltpu.SemaphoreType.DMA for _ in range(NSLOT)],
            [pltpu.SemaphoreType.DMA for _ in range(NSLOT)],
        ],
    )
    def emb(idx_hbm, table_hbm, pos_hbm, out_hbm,
            idx_v, wbuf, pbuf, fsem, gsem, osem):
        wid = lax.axis_index("s") * NC + lax.axis_index("c")
        s_base = wid * s_per_w

        for b in range(B):
            pltpu.sync_copy(
                idx_hbm.at[pl.ds(b * S + s_base, s_per_w)],
                idx_v.at[pl.ds(b * s_per_w, s_per_w)],
            )

        def launch_group(g, slot):
            descs = [pltpu.async_copy(
                pos_hbm.at[pl.ds(s_base + g * CHUNK, CHUNK)],
                pbuf[slot], fsem[slot],
            )]
            for b in range(B):
                descs.append(pltpu.async_copy(
                    table_hbm.at[idx_v.at[pl.ds(b * s_per_w + g * CHUNK, CHUNK)]],
                    wbuf[slot][b], gsem[slot], add=False,
                ))
            return descs

        JU = 8

        def add_group(slot):
            def rbody(r, carry):
                def jbody(jq, c2):
                    base = pl.multiple_of(jq * JU * L, JU * L)
                    for u in range(JU):
                        sl = pl.ds(base + u * L, L)
                        p = pbuf[slot][r, sl]
                        for b in range(B):
                            plsc.addupdate(wbuf[slot][b].at[r, sl], p)
                    return c2

                return lax.fori_loop(0, D // L // JU, jbody, carry)

            lax.fori_loop(0, CHUNK, rbody, 0)

        def store_group(g, slot):
            return [pltpu.async_copy(
                wbuf[slot][b],
                out_hbm.at[pl.ds(b * S + s_base + g * CHUNK, CHUNK)],
                osem[slot],
            ) for b in range(B)]

        pend_g = [None] * NSLOT
        pend_o = [None] * NSLOT
        pend_g[0] = launch_group(0, 0)
        for g in range(n_groups):
            slot = g % NSLOT
            nxt = (g + 1) % NSLOT
            if g + 1 < n_groups:
                if pend_o[nxt] is not None:
                    for d in pend_o[nxt]:
                        d.wait()
                    pend_o[nxt] = None
                pend_g[nxt] = launch_group(g + 1, nxt)
            for d in pend_g[slot]:
                d.wait()
            pend_g[slot] = None
            add_group(slot)
            pend_o[slot] = store_group(g, slot)
        for descs in pend_o:
            if descs is not None:
                for d in descs:
                    d.wait()

    return emb


def kernel(input_ids, word_embeddings, position_embeddings):
    B, S = input_ids.shape
    V, D = word_embeddings.shape
    ids_flat = input_ids.reshape(-1).astype(jnp.int32)
    emb = _make_sc_embed(B, S, V, D)
    out = emb(ids_flat, word_embeddings, position_embeddings)
    return out.reshape(B, S, D)


# speedup vs baseline: 1.0055x; 1.0055x over previous
import functools

import jax
import jax.numpy as jnp
from jax import lax
from jax.experimental import pallas as pl
from jax.experimental.pallas import tpu as pltpu
from jax.experimental.pallas import tpu_sc as plsc


@functools.cache
def _make_sc_embed(B: int, S: int, V: int, D: int):
    info = plsc.get_sparse_core_info()
    NC, NS, L = info.num_cores, info.num_subcores, info.num_lanes
    NW = NC * NS
    assert S % NW == 0
    s_per_w = S // NW
    CHUNK = 32
    HALF = CHUNK // 2
    assert s_per_w % CHUNK == 0
    n_sc = s_per_w // CHUNK
    n_steps = n_sc * B
    mesh = plsc.VectorSubcoreMesh(core_axis_name="c", subcore_axis_name="s")

    @functools.partial(
        pl.kernel,
        mesh=mesh,
        out_type=jax.ShapeDtypeStruct((B * S, D), jnp.float32),
        scratch_types=[
            pltpu.VMEM((B * s_per_w,), jnp.int32),
            pltpu.VMEM((CHUNK, D), jnp.float32),
            pltpu.VMEM((CHUNK, D), jnp.float32),
            pltpu.VMEM((CHUNK, D), jnp.float32),
            pltpu.SemaphoreType.DMA,
            pltpu.SemaphoreType.DMA,
            pltpu.SemaphoreType.DMA,
            pltpu.SemaphoreType.DMA,
            pltpu.SemaphoreType.DMA,
        ],
    )
    def emb(idx_hbm, table_hbm, pos_hbm, out_hbm,
            idx_v, w0, w1, pos_v, g0, g1, o0, o1, psem):
        wid = lax.axis_index("s") * NC + lax.axis_index("c")
        s_base = wid * s_per_w
        wbuf = (w0, w1)
        gsem = (g0, g1)
        osem = (o0, o1)

        for b in range(B):
            pltpu.sync_copy(
                idx_hbm.at[pl.ds(b * S + s_base, s_per_w)],
                idx_v.at[pl.ds(b * s_per_w, s_per_w)],
            )

        def gather(k, buf):
            sc, b = divmod(k, B)
            off = b * s_per_w + sc * CHUNK
            return pltpu.async_copy(
                table_hbm.at[idx_v.at[pl.ds(off, CHUNK)]],
                wbuf[buf], gsem[buf],
            )

        def fill_pos(sc):
            return pltpu.async_copy(
                pos_hbm.at[pl.ds(s_base + sc * CHUNK, CHUNK)], pos_v, psem
            )

        def add_half(buf, h):
            cur = wbuf[buf]

            def body(r, carry):
                for j in range(D // L):
                    sl = pl.ds(j * L, L)
                    plsc.addupdate(cur.at[r, sl], pos_v[r, sl])
                return carry

            lax.fori_loop(h * HALF, (h + 1) * HALF, body, 0)

        def store_half(k, buf, h):
            sc, b = divmod(k, B)
            return pltpu.async_copy(
                wbuf[buf].at[pl.ds(h * HALF, HALF)],
                out_hbm.at[pl.ds(b * S + s_base + sc * CHUNK + h * HALF, HALF)],
                osem[buf],
            )

        pend_pos = fill_pos(0)
        pending_g = gather(0, 0)
        pending_o = [[], []]
        for k in range(n_steps):
            sc, b = divmod(k, B)
            cur = k % 2
            nxt = (k + 1) % 2
            if b == 0:
                pend_pos.wait()
                pend_pos = None
            if k + 1 < n_steps:
                for d in pending_o[nxt]:
                    d.wait()
                pending_o[nxt] = []
                next_g = gather(k + 1, nxt)
            pending_g.wait()
            add_half(cur, 0)
            pending_o[cur] = [store_half(k, cur, 0)]
            add_half(cur, 1)
            pending_o[cur].append(store_half(k, cur, 1))
            if b == B - 1 and sc + 1 < n_sc:
                pend_pos = fill_pos(sc + 1)
            if k + 1 < n_steps:
                pending_g = next_g
        for descs in pending_o:
            for d in descs:
                d.wait()

    return emb


def kernel(input_ids, word_embeddings, position_embeddings):
    B, S = input_ids.shape
    V, D = word_embeddings.shape
    ids_flat = input_ids.reshape(-1).astype(jnp.int32)
    emb = _make_sc_embed(B, S, V, D)
    out = emb(ids_flat, word_embeddings, position_embeddings)
    return out.reshape(B, S, D)
